# Initial kernel scaffold; baseline (speedup 1.0000x reference)
import functools

import jax
import jax.numpy as jnp
from jax import lax
from jax.experimental import pallas as pl
from jax.experimental.pallas import tpu as pltpu
from jax.experimental.pallas import tpu_sc as plsc

N = 10000
E = 320000
C = 128
NEG_SLOPE = 0.2

NC = 2
NS = 16
NW = NC * NS
EPT = E // NW
L = 16
K = 80
GB = EPT // K
NPAD = 10240
CH = NPAD // NS
RPT = N // NS
OB = 125


def _tc_nodes_body(xs, xd, w, a_s, a_d, hs, hd, als, ald):
    hsv = jnp.dot(xs[...], w[...], preferred_element_type=jnp.float32)
    hdv = jnp.dot(xd[...], w[...], preferred_element_type=jnp.float32)
    hs[...] = hsv
    hd[...] = hdv
    als[...] = jnp.sum(hsv * a_s[...], axis=-1, keepdims=True)
    ald[...] = jnp.sum(hdv * a_d[...], axis=-1, keepdims=True)


def _tc_edge_body(ea, we, a_e, ae):
    wv = jnp.dot(we[...], a_e[...].T, preferred_element_type=jnp.float32)
    ae[...] = jnp.dot(ea[...], wv, preferred_element_type=jnp.float32)


def _tc_comb_body(p, hd, b, o):
    o[...] = p[0] + p[1] + hd[...] + b[...]


def _sc_phase1_body(asrc_hbm, adst_hbm, ae_hbm, src_hbm, dst_hbm,
                    ex_hbm, den_hbm,
                    asrc_v, adst_v, src_v, dst_v, ae_v, ex_v, den_v,
                    slab_v, red_v, shared_den):
    cidx = lax.axis_index("c")
    s = lax.axis_index("s")
    wid = s * NC + cidx
    base = pl.multiple_of(wid * EPT, 8)

    pltpu.sync_copy(asrc_hbm, asrc_v)
    pltpu.sync_copy(adst_hbm, adst_v)
    pltpu.sync_copy(src_hbm.at[pl.ds(base, EPT)], src_v)
    pltpu.sync_copy(dst_hbm.at[pl.ds(base, EPT)], dst_v)
    pltpu.sync_copy(ae_hbm.at[pl.ds(base, EPT)], ae_v)

    def zero_step(i, carry):
        den_v[pl.ds(i * L, L)] = jnp.zeros((L,), jnp.float32)
        return carry
    lax.fori_loop(0, NPAD // L, zero_step, 0)

    def edge_step(i, carry):
        sl = pl.ds(i * L, L)
        si = src_v[sl]
        di = dst_v[sl]
        a = plsc.load_gather(asrc_v, [si]) + plsc.load_gather(adst_v, [di])
        a = a + ae_v[sl]
        a = jnp.where(a > 0, a, NEG_SLOPE * a)
        e = jnp.exp(a)
        ex_v[sl] = e
        plsc.addupdate_scatter(den_v, [di], e)
        return carry
    lax.fori_loop(0, EPT // L, edge_step, 0)

    pltpu.sync_copy(ex_v, ex_hbm.at[pl.ds(base, EPT)])

    pltpu.sync_copy(den_v, shared_den.at[s])
    plsc.subcore_barrier()
    col = pl.multiple_of(s * CH, 8)
    for r in range(NS):
        pltpu.sync_copy(shared_den.at[r].at[pl.ds(col, CH)], slab_v.at[r])

    def red_step(j, carry):
        sl = pl.ds(j * L, L)
        acc = slab_v[0, sl]
        for r in range(1, NS):
            acc = acc + slab_v[r, sl]
        red_v[sl] = acc
        return carry
    lax.fori_loop(0, CH // L, red_step, 0)
    pltpu.sync_copy(red_v, den_hbm.at[cidx].at[pl.ds(col, CH)])


def _sc_phase2_body(h_hbm, src3_hbm, dst3_hbm, ex3_hbm, den2_hbm,
                    part_hbm,
                    src_v, dst_v, ex_v, den_v, dent_v, coef_v, rows_v, obuf,
                    acc_sh):
    cidx = lax.axis_index("c")
    s = lax.axis_index("s")
    wid = s * NC + cidx

    pltpu.sync_copy(src3_hbm.at[wid], src_v)
    pltpu.sync_copy(dst3_hbm.at[wid], dst_v)
    pltpu.sync_copy(ex3_hbm.at[wid], ex_v)
    pltpu.sync_copy(den2_hbm.at[0], den_v)
    pltpu.sync_copy(den2_hbm.at[1], dent_v)

    def den_step(i, carry):
        sl = pl.ds(i * L, L)
        den_v[sl] = den_v[sl] + dent_v[sl]
        return carry
    lax.fori_loop(0, NPAD // L, den_step, 0)

    def zrow(i, carry):
        for cc in range(C // L):
            obuf[i, pl.ds(cc * L, L)] = jnp.zeros((L,), jnp.float32)
        return carry
    lax.fori_loop(0, OB, zrow, 0)
    for b in range(RPT // OB):
        pltpu.sync_copy(obuf, acc_sh.at[pl.ds(s * RPT + b * OB, OB)])
    plsc.subcore_barrier()

    def gblock(g, carry):
        pltpu.sync_copy(h_hbm.at[src_v.at[g]], rows_v)
        for j in range(K // L):
            sl = pl.ds(j * L, L)
            di = dst_v[g, sl]
            dv = plsc.load_gather(den_v, [di])
            coef_v[sl] = ex_v[g, sl] / (dv + 1e-16)

        def scale_step(i, carry2):
            cvec = plsc.load_gather(coef_v, [jnp.full((L,), i, jnp.int32)])
            for cc in range(C // L):
                csl = pl.ds(cc * L, L)
                rows_v[i, csl] = rows_v[i, csl] * cvec
            return carry2
        lax.fori_loop(0, K, scale_step, 0)

        pltpu.sync_copy(rows_v, acc_sh.at[dst_v.at[g]], add=True)
        return carry
    lax.fori_loop(0, GB, gblock, 0)

    plsc.subcore_barrier()
    for b in range(RPT // OB):
        r0 = s * RPT + b * OB
        pltpu.sync_copy(acc_sh.at[pl.ds(r0, OB)], obuf)
        pltpu.sync_copy(obuf, part_hbm.at[cidx].at[pl.ds(r0, OB)])


@functools.lru_cache(maxsize=None)
def _build_sc_kernels():
    mesh = plsc.VectorSubcoreMesh(core_axis_name="c", subcore_axis_name="s",
                                  num_cores=NC, num_subcores=NS)
    phase1 = pl.kernel(
        _sc_phase1_body,
        out_type=[jax.ShapeDtypeStruct((E,), jnp.float32),
                  jax.ShapeDtypeStruct((NC, NPAD), jnp.float32)],
        mesh=mesh,
        scratch_types=[
            pltpu.VMEM((N,), jnp.float32),
            pltpu.VMEM((N,), jnp.float32),
            pltpu.VMEM((EPT,), jnp.int32),
            pltpu.VMEM((EPT,), jnp.int32),
            pltpu.VMEM((EPT,), jnp.float32),
            pltpu.VMEM((EPT,), jnp.float32),
            pltpu.VMEM((NPAD,), jnp.float32),
            pltpu.VMEM((NS, CH), jnp.float32),
            pltpu.VMEM((CH,), jnp.float32),
            pltpu.VMEM_SHARED((NS, NPAD), jnp.float32),
        ],
    )
    phase2 = pl.kernel(
        _sc_phase2_body,
        out_type=[jax.ShapeDtypeStruct((NC, N, C), jnp.float32)],
        mesh=mesh,
        scratch_types=[
            pltpu.VMEM((GB, K), jnp.int32),
            pltpu.VMEM((GB, K), jnp.int32),
            pltpu.VMEM((GB, K), jnp.float32),
            pltpu.VMEM((NPAD,), jnp.float32),
            pltpu.VMEM((NPAD,), jnp.float32),
            pltpu.VMEM((K,), jnp.float32),
            pltpu.VMEM((K, C), jnp.float32),
            pltpu.VMEM((OB, C), jnp.float32),
            pltpu.VMEM_SHARED((N, C), jnp.float32),
        ],
    )
    return phase1, phase2


def kernel(x_src, x_dst, edge_index, edge_attr, W_src, W_edge, att_src,
           att_dst, att_edge, bias):
    src = edge_index[0]
    dst = edge_index[1]

    hs, hd, als, ald = pl.pallas_call(
        _tc_nodes_body,
        out_shape=[
            jax.ShapeDtypeStruct((N, C), jnp.float32),
            jax.ShapeDtypeStruct((N, C), jnp.float32),
            jax.ShapeDtypeStruct((N, 1), jnp.float32),
            jax.ShapeDtypeStruct((N, 1), jnp.float32),
        ],
    )(x_src, x_dst, W_src, att_src, att_dst)

    EB = 20000
    ae = pl.pallas_call(
        _tc_edge_body,
        grid=(E // EB,),
        in_specs=[
            pl.BlockSpec((EB, C), lambda i: (i, 0)),
            pl.BlockSpec((C, C), lambda i: (0, 0)),
            pl.BlockSpec((1, C), lambda i: (0, 0)),
        ],
        out_specs=pl.BlockSpec((EB, 1), lambda i: (i, 0)),
        out_shape=jax.ShapeDtypeStruct((E, 1), jnp.float32),
    )(edge_attr, W_edge, att_edge)

    phase1, phase2 = _build_sc_kernels()
    ex, den2 = phase1(als.reshape(N), ald.reshape(N), ae.reshape(E), src, dst)
    part, = phase2(hs, src.reshape(NW, GB, K), dst.reshape(NW, GB, K),
                   ex.reshape(NW, GB, K), den2)

    out = pl.pallas_call(
        _tc_comb_body,
        out_shape=jax.ShapeDtypeStruct((N, C), jnp.float32),
    )(part, hd, bias.reshape(1, C))
    return out


# speedup vs baseline: 6.5641x; 6.5641x over previous
import functools

import jax
import jax.numpy as jnp
from jax import lax
from jax.experimental import pallas as pl
from jax.experimental.pallas import tpu as pltpu
from jax.experimental.pallas import tpu_sc as plsc

N = 10000
E = 320000
C = 128
NEG_SLOPE = 0.2

NC = 2
NS = 16
NW = NC * NS
EPT = E // NW
L = 16
K = 80
NPAD = 10240
CH = NPAD // NS
TW = C // NW
CH2 = 8000
NCHUNK = E // CH2


def _tc_nodes_body(xs, xd, w, a_s, a_d, hs, hd, als, ald):
    hsv = jnp.dot(xs[...], w[...], preferred_element_type=jnp.float32)
    hdv = jnp.dot(xd[...], w[...], preferred_element_type=jnp.float32)
    hs[...] = hsv
    hd[...] = hdv
    als[...] = jnp.sum(hsv * a_s[...], axis=-1, keepdims=True)
    ald[...] = jnp.sum(hdv * a_d[...], axis=-1, keepdims=True)


def _tc_edge_body(ea, we, a_e, ae):
    wv = jnp.dot(we[...], a_e[...].T, preferred_element_type=jnp.float32)
    ae[...] = jnp.dot(ea[...], wv, preferred_element_type=jnp.float32)


def _tc_comb_body(p, hd, b, o):
    o[...] = p[...] + hd[...] + b[...]


def _sc_phase1_body(asrc_hbm, adst_hbm, ae_hbm, src_hbm, dst_hbm,
                    ex_hbm, den_hbm,
                    asrc_v, adst_v, src_v, dst_v, ae_v, ex_v, den_v,
                    slab_v, red_v, shared_den):
    cidx = lax.axis_index("c")
    s = lax.axis_index("s")
    wid = s * NC + cidx
    base = pl.multiple_of(wid * EPT, 8)

    pltpu.sync_copy(asrc_hbm, asrc_v)
    pltpu.sync_copy(adst_hbm, adst_v)
    pltpu.sync_copy(src_hbm.at[pl.ds(base, EPT)], src_v)
    pltpu.sync_copy(dst_hbm.at[pl.ds(base, EPT)], dst_v)
    pltpu.sync_copy(ae_hbm.at[pl.ds(base, EPT)], ae_v)

    def zero_step(i, carry):
        den_v[pl.ds(i * L, L)] = jnp.zeros((L,), jnp.float32)
        return carry
    lax.fori_loop(0, NPAD // L, zero_step, 0)

    def edge_step(i, carry):
        sl = pl.ds(i * L, L)
        si = src_v[sl]
        di = dst_v[sl]
        a = plsc.load_gather(asrc_v, [si]) + plsc.load_gather(adst_v, [di])
        a = a + ae_v[sl]
        a = jnp.where(a > 0, a, NEG_SLOPE * a)
        e = jnp.exp(a)
        ex_v[sl] = e
        plsc.addupdate_scatter(den_v, [di], e)
        return carry
    lax.fori_loop(0, EPT // L, edge_step, 0)

    pltpu.sync_copy(ex_v, ex_hbm.at[pl.ds(base, EPT)])

    pltpu.sync_copy(den_v, shared_den.at[s])
    plsc.subcore_barrier()
    col = pl.multiple_of(s * CH, 8)
    for r in range(NS):
        pltpu.sync_copy(shared_den.at[r].at[pl.ds(col, CH)], slab_v.at[r])

    def red_step(j, carry):
        sl = pl.ds(j * L, L)
        acc = slab_v[0, sl]
        for r in range(1, NS):
            acc = acc + slab_v[r, sl]
        red_v[sl] = acc
        return carry
    lax.fori_loop(0, CH // L, red_step, 0)
    pltpu.sync_copy(red_v, den_hbm.at[cidx].at[pl.ds(col, CH)])


def _sc_phase2_body(htf_hbm, src_hbm, dst_hbm, exd_hbm,
                    outf_hbm,
                    h_v, out_v, den_v, dent_v, src_v, dst_v, ex_v):
    cidx = lax.axis_index("c")
    s = lax.axis_index("s")
    w = s * NC + cidx

    hbase = pl.multiple_of(w * (N * TW), 8)
    pltpu.sync_copy(htf_hbm.at[pl.ds(hbase, N * TW)], h_v)
    d0 = pl.multiple_of(cidx * NPAD, 8)
    d1 = pl.multiple_of((1 - cidx) * NPAD, 8)
    pltpu.sync_copy(exd_hbm.at[pl.ds(d0, NPAD)], den_v)
    pltpu.sync_copy(exd_hbm.at[pl.ds(d1, NPAD)], dent_v)

    def den_step(i, carry):
        sl = pl.ds(i * L, L)
        den_v[sl] = den_v[sl] + dent_v[sl]
        return carry
    lax.fori_loop(0, NPAD // L, den_step, 0)

    def zero_step(i, carry):
        out_v[pl.ds(i * L, L)] = jnp.zeros((L,), jnp.float32)
        return carry
    lax.fori_loop(0, (N * TW) // L, zero_step, 0)

    def chunk_step(cc, carry):
        ci = lax.rem(cc + w, jnp.int32(NCHUNK))
        cbase = pl.multiple_of(ci * CH2, 8)
        pltpu.sync_copy(src_hbm.at[pl.ds(cbase, CH2)], src_v)
        pltpu.sync_copy(dst_hbm.at[pl.ds(cbase, CH2)], dst_v)
        ebase = pl.multiple_of(NC * NPAD + ci * CH2, 8)
        pltpu.sync_copy(exd_hbm.at[pl.ds(ebase, CH2)], ex_v)

        def edge_step(i, carry2):
            sl = pl.ds(i * L, L)
            si = src_v[sl]
            di = dst_v[sl]
            dv = plsc.load_gather(den_v, [di])
            cf = ex_v[sl] / (dv + 1e-16)
            si4 = si * TW
            di4 = di * TW
            for c in range(TW):
                hv = plsc.load_gather(h_v, [si4 + c])
                plsc.addupdate_scatter(out_v, [di4 + c], hv * cf)
            return carry2
        lax.fori_loop(0, CH2 // L, edge_step, 0)
        return carry
    lax.fori_loop(0, NCHUNK, chunk_step, 0)

    pltpu.sync_copy(out_v, outf_hbm.at[pl.ds(hbase, N * TW)])


@functools.lru_cache(maxsize=None)
def _build_sc_kernels():
    mesh = plsc.VectorSubcoreMesh(core_axis_name="c", subcore_axis_name="s",
                                  num_cores=NC, num_subcores=NS)
    params1 = pltpu.CompilerParams(needs_layout_passes=False,
                                   use_tc_tiling_on_sc=False)
    phase1 = pl.kernel(
        _sc_phase1_body,
        out_type=[jax.ShapeDtypeStruct((E,), jnp.float32),
                  jax.ShapeDtypeStruct((NC, NPAD), jnp.float32)],
        mesh=mesh,
        compiler_params=params1,
        scratch_types=[
            pltpu.VMEM((N,), jnp.float32),
            pltpu.VMEM((N,), jnp.float32),
            pltpu.VMEM((EPT,), jnp.int32),
            pltpu.VMEM((EPT,), jnp.int32),
            pltpu.VMEM((EPT,), jnp.float32),
            pltpu.VMEM((EPT,), jnp.float32),
            pltpu.VMEM((NPAD,), jnp.float32),
            pltpu.VMEM((NS, CH), jnp.float32),
            pltpu.VMEM((CH,), jnp.float32),
            pltpu.VMEM_SHARED((NS, NPAD), jnp.float32),
        ],
    )
    phase2 = pl.kernel(
        _sc_phase2_body,
        out_type=[jax.ShapeDtypeStruct((NW * N * TW,), jnp.float32)],
        mesh=mesh,
        compiler_params=params1,
        scratch_types=[
            pltpu.VMEM((N * TW,), jnp.float32),
            pltpu.VMEM((N * TW,), jnp.float32),
            pltpu.VMEM((NPAD,), jnp.float32),
            pltpu.VMEM((NPAD,), jnp.float32),
            pltpu.VMEM((CH2,), jnp.int32),
            pltpu.VMEM((CH2,), jnp.int32),
            pltpu.VMEM((CH2,), jnp.float32),
        ],
    )
    return phase1, phase2


def kernel(x_src, x_dst, edge_index, edge_attr, W_src, W_edge, att_src,
           att_dst, att_edge, bias):
    src = edge_index[0]
    dst = edge_index[1]

    hs, hd, als, ald = pl.pallas_call(
        _tc_nodes_body,
        out_shape=[
            jax.ShapeDtypeStruct((N, C), jnp.float32),
            jax.ShapeDtypeStruct((N, C), jnp.float32),
            jax.ShapeDtypeStruct((N, 1), jnp.float32),
            jax.ShapeDtypeStruct((N, 1), jnp.float32),
        ],
    )(x_src, x_dst, W_src, att_src, att_dst)

    EB = 20000
    ae = pl.pallas_call(
        _tc_edge_body,
        grid=(E // EB,),
        in_specs=[
            pl.BlockSpec((EB, C), lambda i: (i, 0)),
            pl.BlockSpec((C, C), lambda i: (0, 0)),
            pl.BlockSpec((1, C), lambda i: (0, 0)),
        ],
        out_specs=pl.BlockSpec((EB, 1), lambda i: (i, 0)),
        out_shape=jax.ShapeDtypeStruct((E, 1), jnp.float32),
    )(edge_attr, W_edge, att_edge)

    phase1, phase2 = _build_sc_kernels()
    ex, den2 = phase1(als.reshape(N), ald.reshape(N), ae.reshape(E), src, dst)
    exd = jnp.concatenate([den2.reshape(NC * NPAD), ex])
    htf = hs.reshape(N, NW, TW).transpose(1, 0, 2).reshape(NW * N * TW)
    partf, = phase2(htf, src, dst, exd)
    part = partf.reshape(NW, N, TW).transpose(1, 0, 2).reshape(N, C)

    out = pl.pallas_call(
        _tc_comb_body,
        out_shape=jax.ShapeDtypeStruct((N, C), jnp.float32),
    )(part, hd, bias.reshape(1, C))
    return out


# speedup vs baseline: 11.8268x; 1.8018x over previous
import functools

import jax
import jax.numpy as jnp
from jax import lax
from jax.experimental import pallas as pl
from jax.experimental.pallas import tpu as pltpu
from jax.experimental.pallas import tpu_sc as plsc

N = 10000
E = 320000
C = 128
NEG_SLOPE = 0.2

NC = 2
NS = 16
NW = NC * NS
EPT = E // NW
L = 16
K = 80
NPAD = 10240
CH = NPAD // NS
TW = C // NW
CH2 = 8000
NCHUNK = E // CH2


def _tc_nodes_body(xs, xd, w, a_s, a_d, hst, hd, als, ald):
    hstv = lax.dot_general(w[...], xs[...], (((0,), (1,)), ((), ())),
                           preferred_element_type=jnp.float32)
    hst[...] = hstv
    hdv = jnp.dot(xd[...], w[...], preferred_element_type=jnp.float32)
    hd[...] = hdv
    als[...] = jnp.sum(hstv * a_s[...].reshape(C, 1), axis=0, keepdims=True)
    ald[...] = jnp.sum(hdv * a_d[...], axis=-1, keepdims=True)


def _tc_edge_body(ea, we, a_e, ae):
    wv = jnp.dot(we[...], a_e[...].T, preferred_element_type=jnp.float32)
    ae[...] = jnp.dot(ea[...], wv, preferred_element_type=jnp.float32)


def _tc_comb_body(p, hd, b, o):
    o[...] = p[...].T + hd[...] + b[...]


def _sc_phase1_body(asrc_hbm, adst_hbm, ae_hbm, src_hbm, dst_hbm,
                    ex_hbm, den_hbm,
                    asrc_v, adst_v, src_v, dst_v, ae_v, ex_v, den_v,
                    slab_v, red_v, shared_den):
    cidx = lax.axis_index("c")
    s = lax.axis_index("s")
    wid = s * NC + cidx
    base = pl.multiple_of(wid * EPT, 8)

    pltpu.sync_copy(asrc_hbm, asrc_v)
    pltpu.sync_copy(adst_hbm, adst_v)
    pltpu.sync_copy(src_hbm.at[pl.ds(base, EPT)], src_v)
    pltpu.sync_copy(dst_hbm.at[pl.ds(base, EPT)], dst_v)
    pltpu.sync_copy(ae_hbm.at[pl.ds(base, EPT)], ae_v)

    def zero_step(i, carry):
        den_v[pl.ds(i * L, L)] = jnp.zeros((L,), jnp.float32)
        return carry
    lax.fori_loop(0, NPAD // L, zero_step, 0)

    def edge_step(i, carry):
        sl = pl.ds(i * L, L)
        si = src_v[sl]
        di = dst_v[sl]
        a = plsc.load_gather(asrc_v, [si]) + plsc.load_gather(adst_v, [di])
        a = a + ae_v[sl]
        a = jnp.where(a > 0, a, NEG_SLOPE * a)
        e = jnp.exp(a)
        ex_v[sl] = e
        plsc.addupdate_scatter(den_v, [di], e)
        return carry
    lax.fori_loop(0, EPT // L, edge_step, 0)

    pltpu.sync_copy(ex_v, ex_hbm.at[pl.ds(base, EPT)])

    pltpu.sync_copy(den_v, shared_den.at[s])
    plsc.subcore_barrier()
    col = pl.multiple_of(s * CH, 8)
    for r in range(NS):
        pltpu.sync_copy(shared_den.at[r].at[pl.ds(col, CH)], slab_v.at[r])

    def red_step(j, carry):
        sl = pl.ds(j * L, L)
        acc = slab_v[0, sl]
        for r in range(1, NS):
            acc = acc + slab_v[r, sl]
        red_v[sl] = acc
        return carry
    lax.fori_loop(0, CH // L, red_step, 0)
    pltpu.sync_copy(red_v, den_hbm.at[cidx].at[pl.ds(col, CH)])


def _sc_phase1b_body(dst_hbm, exd_hbm,
                     coef_hbm,
                     dst_v, ex_v, cf_v, den_v, dent_v):
    cidx = lax.axis_index("c")
    s = lax.axis_index("s")
    wid = s * NC + cidx
    base = pl.multiple_of(wid * EPT, 8)

    pltpu.sync_copy(dst_hbm.at[pl.ds(base, EPT)], dst_v)
    ebase = pl.multiple_of(NC * NPAD + wid * EPT, 8)
    pltpu.sync_copy(exd_hbm.at[pl.ds(ebase, EPT)], ex_v)
    d0 = pl.multiple_of(cidx * NPAD, 8)
    d1 = pl.multiple_of((1 - cidx) * NPAD, 8)
    pltpu.sync_copy(exd_hbm.at[pl.ds(d0, NPAD)], den_v)
    pltpu.sync_copy(exd_hbm.at[pl.ds(d1, NPAD)], dent_v)

    def den_step(i, carry):
        sl = pl.ds(i * L, L)
        den_v[sl] = den_v[sl] + dent_v[sl]
        return carry
    lax.fori_loop(0, NPAD // L, den_step, 0)

    def edge_step(i, carry):
        sl = pl.ds(i * L, L)
        di = dst_v[sl]
        dv = plsc.load_gather(den_v, [di])
        cf_v[sl] = ex_v[sl] / (dv + 1e-16)
        return carry
    lax.fori_loop(0, EPT // L, edge_step, 0)
    pltpu.sync_copy(cf_v, coef_hbm.at[pl.ds(base, EPT)])


def _sc_phase2_body(htf_hbm, src_hbm, dst_hbm, coef_hbm,
                    outf_hbm,
                    h_v, out_v, src_v, dst_v, cf_v):
    cidx = lax.axis_index("c")
    s = lax.axis_index("s")
    w = s * NC + cidx

    hbase = pl.multiple_of(w * (TW * N), 8)
    pltpu.sync_copy(htf_hbm.at[pl.ds(hbase, TW * N)], h_v)

    def zero_step(i, carry):
        out_v[pl.ds(i * L, L)] = jnp.zeros((L,), jnp.float32)
        return carry
    lax.fori_loop(0, (TW * N) // L, zero_step, 0)

    def chunk_step(cc, carry):
        ci = lax.rem(cc + w, jnp.int32(NCHUNK))
        cbase = pl.multiple_of(ci * CH2, 8)
        pltpu.sync_copy(src_hbm.at[pl.ds(cbase, CH2)], src_v)
        pltpu.sync_copy(dst_hbm.at[pl.ds(cbase, CH2)], dst_v)
        pltpu.sync_copy(coef_hbm.at[pl.ds(cbase, CH2)], cf_v)

        def edge_step(i, carry2):
            sl = pl.ds(i * L, L)
            si = src_v[sl]
            di = dst_v[sl]
            cf = cf_v[sl]
            for c in range(TW):
                hv = plsc.load_gather(h_v, [si + c * N])
                plsc.addupdate_scatter(out_v, [di + c * N], hv * cf)
            return carry2
        lax.fori_loop(0, CH2 // L, edge_step, 0)
        return carry
    lax.fori_loop(0, NCHUNK, chunk_step, 0)

    pltpu.sync_copy(out_v, outf_hbm.at[pl.ds(hbase, TW * N)])


@functools.lru_cache(maxsize=None)
def _build_sc_kernels():
    mesh = plsc.VectorSubcoreMesh(core_axis_name="c", subcore_axis_name="s",
                                  num_cores=NC, num_subcores=NS)
    params1 = pltpu.CompilerParams(needs_layout_passes=False,
                                   use_tc_tiling_on_sc=False)
    phase1 = pl.kernel(
        _sc_phase1_body,
        out_type=[jax.ShapeDtypeStruct((E,), jnp.float32),
                  jax.ShapeDtypeStruct((NC, NPAD), jnp.float32)],
        mesh=mesh,
        compiler_params=params1,
        scratch_types=[
            pltpu.VMEM((N,), jnp.float32),
            pltpu.VMEM((N,), jnp.float32),
            pltpu.VMEM((EPT,), jnp.int32),
            pltpu.VMEM((EPT,), jnp.int32),
            pltpu.VMEM((EPT,), jnp.float32),
            pltpu.VMEM((EPT,), jnp.float32),
            pltpu.VMEM((NPAD,), jnp.float32),
            pltpu.VMEM((NS, CH), jnp.float32),
            pltpu.VMEM((CH,), jnp.float32),
            pltpu.VMEM_SHARED((NS, NPAD), jnp.float32),
        ],
    )
    phase1b = pl.kernel(
        _sc_phase1b_body,
        out_type=[jax.ShapeDtypeStruct((E,), jnp.float32)],
        mesh=mesh,
        compiler_params=params1,
        scratch_types=[
            pltpu.VMEM((EPT,), jnp.int32),
            pltpu.VMEM((EPT,), jnp.float32),
            pltpu.VMEM((EPT,), jnp.float32),
            pltpu.VMEM((NPAD,), jnp.float32),
            pltpu.VMEM((NPAD,), jnp.float32),
        ],
    )
    phase2 = pl.kernel(
        _sc_phase2_body,
        out_type=[jax.ShapeDtypeStruct((C * N,), jnp.float32)],
        mesh=mesh,
        compiler_params=params1,
        scratch_types=[
            pltpu.VMEM((TW * N,), jnp.float32),
            pltpu.VMEM((TW * N,), jnp.float32),
            pltpu.VMEM((CH2,), jnp.int32),
            pltpu.VMEM((CH2,), jnp.int32),
            pltpu.VMEM((CH2,), jnp.float32),
        ],
    )
    return phase1, phase1b, phase2


def kernel(x_src, x_dst, edge_index, edge_attr, W_src, W_edge, att_src,
           att_dst, att_edge, bias):
    src = edge_index[0]
    dst = edge_index[1]

    hst, hd, als, ald = pl.pallas_call(
        _tc_nodes_body,
        out_shape=[
            jax.ShapeDtypeStruct((C, N), jnp.float32),
            jax.ShapeDtypeStruct((N, C), jnp.float32),
            jax.ShapeDtypeStruct((1, N), jnp.float32),
            jax.ShapeDtypeStruct((N, 1), jnp.float32),
        ],
    )(x_src, x_dst, W_src, att_src, att_dst)

    EB = 20000
    ae = pl.pallas_call(
        _tc_edge_body,
        grid=(E // EB,),
        in_specs=[
            pl.BlockSpec((EB, C), lambda i: (i, 0)),
            pl.BlockSpec((C, C), lambda i: (0, 0)),
            pl.BlockSpec((1, C), lambda i: (0, 0)),
        ],
        out_specs=pl.BlockSpec((EB, 1), lambda i: (i, 0)),
        out_shape=jax.ShapeDtypeStruct((E, 1), jnp.float32),
    )(edge_attr, W_edge, att_edge)

    phase1, phase1b, phase2 = _build_sc_kernels()
    ex, den2 = phase1(als.reshape(N), ald.reshape(N), ae.reshape(E), src, dst)
    exd = jnp.concatenate([den2.reshape(NC * NPAD), ex])
    coef, = phase1b(dst, exd)
    partf, = phase2(hst.reshape(C * N), src, dst, coef)

    out = pl.pallas_call(
        _tc_comb_body,
        out_shape=jax.ShapeDtypeStruct((N, C), jnp.float32),
    )(partf.reshape(C, N), hd, bias.reshape(1, C))
    return out


# speedup vs baseline: 19.5897x; 1.6564x over previous
import functools

import jax
import jax.numpy as jnp
from jax import lax
from jax.experimental import pallas as pl
from jax.experimental.pallas import tpu as pltpu
from jax.experimental.pallas import tpu_sc as plsc

N = 10000
E = 320000
C = 128
NEG_SLOPE = 0.2

NC = 2
NS = 16
NW = NC * NS
EPT = E // NW
L = 16
K = 80
NPAD = 10240
CH = NPAD // NS
TW = C // NW
CH2 = 8000
NCHUNK = E // CH2


def _tc_nodes_body(xs, xd, w, a_s, a_d, hst, hd, als, ald):
    hstv = lax.dot_general(w[...], xs[...], (((0,), (1,)), ((), ())),
                           preferred_element_type=jnp.float32)
    hst[...] = hstv
    hdv = jnp.dot(xd[...], w[...], preferred_element_type=jnp.float32)
    hd[...] = hdv
    als[...] = jnp.sum(hstv * a_s[...].reshape(C, 1), axis=0, keepdims=True)
    ald[...] = jnp.sum(hdv * a_d[...], axis=-1, keepdims=True)


def _tc_edge_body(ea, we, a_e, ae):
    wv = jnp.dot(we[...], a_e[...].T, preferred_element_type=jnp.float32)
    ae[...] = jnp.dot(ea[...], wv, preferred_element_type=jnp.float32)


def _tc_comb_body(p, hd, b, o):
    o[...] = p[...].T + hd[...] + b[...]


def _sc_phase1_body(asrc_hbm, adst_hbm, ae_hbm, src_hbm, dst_hbm,
                    ex_hbm, den_hbm,
                    asrc_v, adst_v, src_v, dst_v, ae_v, ex_v, den_v,
                    slab_v, red_v, shared_den):
    cidx = lax.axis_index("c")
    s = lax.axis_index("s")
    wid = s * NC + cidx
    base = pl.multiple_of(wid * EPT, 8)

    pltpu.sync_copy(asrc_hbm, asrc_v)
    pltpu.sync_copy(adst_hbm, adst_v)
    pltpu.sync_copy(src_hbm.at[pl.ds(base, EPT)], src_v)
    pltpu.sync_copy(dst_hbm.at[pl.ds(base, EPT)], dst_v)
    pltpu.sync_copy(ae_hbm.at[pl.ds(base, EPT)], ae_v)

    def zero_step(i, carry):
        den_v[pl.ds(i * L, L)] = jnp.zeros((L,), jnp.float32)
        return carry
    lax.fori_loop(0, NPAD // L, zero_step, 0)

    def edge_step(i, carry):
        sl = pl.ds(i * L, L)
        si = src_v[sl]
        di = dst_v[sl]
        a = plsc.load_gather(asrc_v, [si]) + plsc.load_gather(adst_v, [di])
        a = a + ae_v[sl]
        a = jnp.where(a > 0, a, NEG_SLOPE * a)
        e = jnp.exp(a)
        ex_v[sl] = e
        plsc.addupdate_scatter(den_v, [di], e)
        return carry
    lax.fori_loop(0, EPT // L, edge_step, 0)

    pltpu.sync_copy(ex_v, ex_hbm.at[pl.ds(base, EPT)])

    pltpu.sync_copy(den_v, shared_den.at[s])
    plsc.subcore_barrier()
    col = pl.multiple_of(s * CH, 8)
    for r in range(NS):
        pltpu.sync_copy(shared_den.at[r].at[pl.ds(col, CH)], slab_v.at[r])

    def red_step(j, carry):
        sl = pl.ds(j * L, L)
        acc = slab_v[0, sl]
        for r in range(1, NS):
            acc = acc + slab_v[r, sl]
        red_v[sl] = acc
        return carry
    lax.fori_loop(0, CH // L, red_step, 0)
    pltpu.sync_copy(red_v, den_hbm.at[cidx].at[pl.ds(col, CH)])


def _sc_phase1b_body(dst_hbm, exd_hbm,
                     coef_hbm,
                     dst_v, ex_v, cf_v, den_v, dent_v):
    cidx = lax.axis_index("c")
    s = lax.axis_index("s")
    wid = s * NC + cidx
    base = pl.multiple_of(wid * EPT, 8)

    pltpu.sync_copy(dst_hbm.at[pl.ds(base, EPT)], dst_v)
    ebase = pl.multiple_of(NC * NPAD + wid * EPT, 8)
    pltpu.sync_copy(exd_hbm.at[pl.ds(ebase, EPT)], ex_v)
    d0 = pl.multiple_of(cidx * NPAD, 8)
    d1 = pl.multiple_of((1 - cidx) * NPAD, 8)
    pltpu.sync_copy(exd_hbm.at[pl.ds(d0, NPAD)], den_v)
    pltpu.sync_copy(exd_hbm.at[pl.ds(d1, NPAD)], dent_v)

    def den_step(i, carry):
        sl = pl.ds(i * L, L)
        den_v[sl] = den_v[sl] + dent_v[sl]
        return carry
    lax.fori_loop(0, NPAD // L, den_step, 0)

    def edge_step(i, carry):
        sl = pl.ds(i * L, L)
        di = dst_v[sl]
        dv = plsc.load_gather(den_v, [di])
        cf_v[sl] = ex_v[sl] / (dv + 1e-16)
        return carry
    lax.fori_loop(0, EPT // L, edge_step, 0)
    pltpu.sync_copy(cf_v, coef_hbm.at[pl.ds(base, EPT)])


def _sc_phase2_body(htf_hbm, src_hbm, dst_hbm, coef_hbm,
                    outf_hbm,
                    h_v, out_v, src_v, dst_v, cf_v):
    cidx = lax.axis_index("c")
    s = lax.axis_index("s")
    w = s * NC + cidx

    hbase = pl.multiple_of(w * (TW * N), 8)
    pltpu.sync_copy(htf_hbm.at[pl.ds(hbase, TW * N)], h_v)

    def zero_step(i, carry):
        out_v[pl.ds(i * L, L)] = jnp.zeros((L,), jnp.float32)
        return carry
    lax.fori_loop(0, (TW * N) // L, zero_step, 0)

    hrefs = [h_v.at[pl.ds(c * N, N)] for c in range(TW)]
    orefs = [out_v.at[pl.ds(c * N, N)] for c in range(TW)]

    def chunk_step(cc, carry):
        ci = lax.rem(cc + w, jnp.int32(NCHUNK))
        cbase = pl.multiple_of(ci * CH2, 8)
        pltpu.sync_copy(src_hbm.at[pl.ds(cbase, CH2)], src_v)
        pltpu.sync_copy(dst_hbm.at[pl.ds(cbase, CH2)], dst_v)
        pltpu.sync_copy(coef_hbm.at[pl.ds(cbase, CH2)], cf_v)

        @plsc.parallel_loop(0, CH2 // L, unroll=4)
        def edge_step(i):
            sl = pl.ds(i * L, L)
            si = src_v[sl]
            di = dst_v[sl]
            cf = cf_v[sl]
            for c in range(TW):
                hv = plsc.load_gather(hrefs[c], [si])
                plsc.addupdate_scatter(orefs[c], [di], hv * cf)
        return carry
    lax.fori_loop(0, NCHUNK, chunk_step, 0)

    pltpu.sync_copy(out_v, outf_hbm.at[pl.ds(hbase, TW * N)])


@functools.lru_cache(maxsize=None)
def _build_sc_kernels():
    mesh = plsc.VectorSubcoreMesh(core_axis_name="c", subcore_axis_name="s",
                                  num_cores=NC, num_subcores=NS)
    params1 = pltpu.CompilerParams(needs_layout_passes=False,
                                   use_tc_tiling_on_sc=False)
    phase1 = pl.kernel(
        _sc_phase1_body,
        out_type=[jax.ShapeDtypeStruct((E,), jnp.float32),
                  jax.ShapeDtypeStruct((NC, NPAD), jnp.float32)],
        mesh=mesh,
        compiler_params=params1,
        scratch_types=[
            pltpu.VMEM((N,), jnp.float32),
            pltpu.VMEM((N,), jnp.float32),
            pltpu.VMEM((EPT,), jnp.int32),
            pltpu.VMEM((EPT,), jnp.int32),
            pltpu.VMEM((EPT,), jnp.float32),
            pltpu.VMEM((EPT,), jnp.float32),
            pltpu.VMEM((NPAD,), jnp.float32),
            pltpu.VMEM((NS, CH), jnp.float32),
            pltpu.VMEM((CH,), jnp.float32),
            pltpu.VMEM_SHARED((NS, NPAD), jnp.float32),
        ],
    )
    phase1b = pl.kernel(
        _sc_phase1b_body,
        out_type=[jax.ShapeDtypeStruct((E,), jnp.float32)],
        mesh=mesh,
        compiler_params=params1,
        scratch_types=[
            pltpu.VMEM((EPT,), jnp.int32),
            pltpu.VMEM((EPT,), jnp.float32),
            pltpu.VMEM((EPT,), jnp.float32),
            pltpu.VMEM((NPAD,), jnp.float32),
            pltpu.VMEM((NPAD,), jnp.float32),
        ],
    )
    phase2 = pl.kernel(
        _sc_phase2_body,
        out_type=[jax.ShapeDtypeStruct((C * N,), jnp.float32)],
        mesh=mesh,
        compiler_params=params1,
        scratch_types=[
            pltpu.VMEM((TW * N,), jnp.float32),
            pltpu.VMEM((TW * N,), jnp.float32),
            pltpu.VMEM((CH2,), jnp.int32),
            pltpu.VMEM((CH2,), jnp.int32),
            pltpu.VMEM((CH2,), jnp.float32),
        ],
    )
    return phase1, phase1b, phase2


def kernel(x_src, x_dst, edge_index, edge_attr, W_src, W_edge, att_src,
           att_dst, att_edge, bias):
    src = edge_index[0]
    dst = edge_index[1]

    hst, hd, als, ald = pl.pallas_call(
        _tc_nodes_body,
        out_shape=[
            jax.ShapeDtypeStruct((C, N), jnp.float32),
            jax.ShapeDtypeStruct((N, C), jnp.float32),
            jax.ShapeDtypeStruct((1, N), jnp.float32),
            jax.ShapeDtypeStruct((N, 1), jnp.float32),
        ],
    )(x_src, x_dst, W_src, att_src, att_dst)

    EB = 20000
    ae = pl.pallas_call(
        _tc_edge_body,
        grid=(E // EB,),
        in_specs=[
            pl.BlockSpec((EB, C), lambda i: (i, 0)),
            pl.BlockSpec((C, C), lambda i: (0, 0)),
            pl.BlockSpec((1, C), lambda i: (0, 0)),
        ],
        out_specs=pl.BlockSpec((EB, 1), lambda i: (i, 0)),
        out_shape=jax.ShapeDtypeStruct((E, 1), jnp.float32),
    )(edge_attr, W_edge, att_edge)

    phase1, phase1b, phase2 = _build_sc_kernels()
    ex, den2 = phase1(als.reshape(N), ald.reshape(N), ae.reshape(E), src, dst)
    exd = jnp.concatenate([den2.reshape(NC * NPAD), ex])
    coef, = phase1b(dst, exd)
    partf, = phase2(hst.reshape(C * N), src, dst, coef)

    out = pl.pallas_call(
        _tc_comb_body,
        out_shape=jax.ShapeDtypeStruct((N, C), jnp.float32),
    )(partf.reshape(C, N), hd, bias.reshape(1, C))
    return out


# speedup vs baseline: 20.2238x; 1.0324x over previous
import functools

import jax
import jax.numpy as jnp
from jax import lax
from jax.experimental import pallas as pl
from jax.experimental.pallas import tpu as pltpu
from jax.experimental.pallas import tpu_sc as plsc

N = 10000
E = 320000
C = 128
NEG_SLOPE = 0.2

NC = 2
NS = 16
NW = NC * NS
EPT = E // NW
L = 16
K = 80
NPAD = 10240
CH = NPAD // NS
TW = C // NW
CH2 = 8000
NCHUNK = E // CH2


def _tc_nodes_body(xs, xd, w, a_s, a_d, hst, hd, als, ald):
    hstv = lax.dot_general(w[...], xs[...], (((0,), (1,)), ((), ())),
                           preferred_element_type=jnp.float32)
    hst[...] = hstv
    hdv = jnp.dot(xd[...], w[...], preferred_element_type=jnp.float32)
    hd[...] = hdv
    als[...] = jnp.sum(hstv * a_s[...].reshape(C, 1), axis=0, keepdims=True)
    ald[...] = jnp.sum(hdv * a_d[...], axis=-1, keepdims=True)


def _tc_edge_body(ea, we, a_e, ae):
    wv = jnp.dot(we[...], a_e[...].T, preferred_element_type=jnp.float32)
    ae[...] = jnp.dot(ea[...], wv, preferred_element_type=jnp.float32)


def _tc_comb_body(p, hd, b, o):
    o[...] = p[...].T + hd[...] + b[...]


def _sc_phase1_body(asrc_hbm, adst_hbm, ae_hbm, src_hbm, dst_hbm,
                    ex_hbm, den_hbm,
                    asrc_v, adst_v, src_v, dst_v, ae_v, ex_v, den_v,
                    slab_v, red_v, shared_den):
    cidx = lax.axis_index("c")
    s = lax.axis_index("s")
    wid = s * NC + cidx
    base = pl.multiple_of(wid * EPT, 8)

    pltpu.sync_copy(asrc_hbm, asrc_v)
    pltpu.sync_copy(adst_hbm, adst_v)
    pltpu.sync_copy(src_hbm.at[pl.ds(base, EPT)], src_v)
    pltpu.sync_copy(dst_hbm.at[pl.ds(base, EPT)], dst_v)
    pltpu.sync_copy(ae_hbm.at[pl.ds(base, EPT)], ae_v)

    @plsc.parallel_loop(0, NPAD // L, unroll=4)
    def zero_step(i):
        den_v[pl.ds(i * L, L)] = jnp.zeros((L,), jnp.float32)

    @plsc.parallel_loop(0, EPT // L, unroll=4)
    def edge_step(i):
        sl = pl.ds(i * L, L)
        si = src_v[sl]
        di = dst_v[sl]
        a = plsc.load_gather(asrc_v, [si]) + plsc.load_gather(adst_v, [di])
        a = a + ae_v[sl]
        a = jnp.where(a > 0, a, NEG_SLOPE * a)
        e = jnp.exp(a)
        ex_v[sl] = e
        plsc.addupdate_scatter(den_v, [di], e)

    pltpu.sync_copy(ex_v, ex_hbm.at[pl.ds(base, EPT)])

    pltpu.sync_copy(den_v, shared_den.at[s])
    plsc.subcore_barrier()
    col = pl.multiple_of(s * CH, 8)
    for r in range(NS):
        pltpu.sync_copy(shared_den.at[r].at[pl.ds(col, CH)], slab_v.at[r])

    @plsc.parallel_loop(0, CH // L, unroll=2)
    def red_step(j):
        sl = pl.ds(j * L, L)
        acc = slab_v[0, sl]
        for r in range(1, NS):
            acc = acc + slab_v[r, sl]
        red_v[sl] = acc
    pltpu.sync_copy(red_v, den_hbm.at[cidx].at[pl.ds(col, CH)])


def _sc_phase1b_body(dst_hbm, exd_hbm,
                     coef_hbm,
                     dst_v, ex_v, cf_v, den_v, dent_v):
    cidx = lax.axis_index("c")
    s = lax.axis_index("s")
    wid = s * NC + cidx
    base = pl.multiple_of(wid * EPT, 8)

    pltpu.sync_copy(dst_hbm.at[pl.ds(base, EPT)], dst_v)
    ebase = pl.multiple_of(NC * NPAD + wid * EPT, 8)
    pltpu.sync_copy(exd_hbm.at[pl.ds(ebase, EPT)], ex_v)
    d0 = pl.multiple_of(cidx * NPAD, 8)
    d1 = pl.multiple_of((1 - cidx) * NPAD, 8)
    pltpu.sync_copy(exd_hbm.at[pl.ds(d0, NPAD)], den_v)
    pltpu.sync_copy(exd_hbm.at[pl.ds(d1, NPAD)], dent_v)

    @plsc.parallel_loop(0, NPAD // L, unroll=4)
    def den_step(i):
        sl = pl.ds(i * L, L)
        den_v[sl] = den_v[sl] + dent_v[sl]

    @plsc.parallel_loop(0, EPT // L, unroll=4)
    def edge_step(i):
        sl = pl.ds(i * L, L)
        di = dst_v[sl]
        dv = plsc.load_gather(den_v, [di])
        cf_v[sl] = ex_v[sl] / (dv + 1e-16)
    pltpu.sync_copy(cf_v, coef_hbm.at[pl.ds(base, EPT)])


def _sc_phase2_body(htf_hbm, src_hbm, dst_hbm, coef_hbm,
                    outf_hbm,
                    h_v, out_v, src_v, dst_v, cf_v):
    cidx = lax.axis_index("c")
    s = lax.axis_index("s")
    w = s * NC + cidx

    hbase = pl.multiple_of(w * (TW * N), 8)
    pltpu.sync_copy(htf_hbm.at[pl.ds(hbase, TW * N)], h_v)

    @plsc.parallel_loop(0, (TW * N) // L, unroll=4)
    def zero_step(i):
        out_v[pl.ds(i * L, L)] = jnp.zeros((L,), jnp.float32)

    hrefs = [h_v.at[pl.ds(c * N, N)] for c in range(TW)]
    orefs = [out_v.at[pl.ds(c * N, N)] for c in range(TW)]

    def chunk_step(cc, carry):
        ci = lax.rem(cc + w, jnp.int32(NCHUNK))
        cbase = pl.multiple_of(ci * CH2, 8)
        pltpu.sync_copy(src_hbm.at[pl.ds(cbase, CH2)], src_v)
        pltpu.sync_copy(dst_hbm.at[pl.ds(cbase, CH2)], dst_v)
        pltpu.sync_copy(coef_hbm.at[pl.ds(cbase, CH2)], cf_v)

        @plsc.parallel_loop(0, CH2 // L, unroll=8)
        def edge_step(i):
            sl = pl.ds(i * L, L)
            si = src_v[sl]
            di = dst_v[sl]
            cf = cf_v[sl]
            for c in range(TW):
                hv = plsc.load_gather(hrefs[c], [si])
                plsc.addupdate_scatter(orefs[c], [di], hv * cf)
        return carry
    lax.fori_loop(0, NCHUNK, chunk_step, 0)

    pltpu.sync_copy(out_v, outf_hbm.at[pl.ds(hbase, TW * N)])


@functools.lru_cache(maxsize=None)
def _build_sc_kernels():
    mesh = plsc.VectorSubcoreMesh(core_axis_name="c", subcore_axis_name="s",
                                  num_cores=NC, num_subcores=NS)
    params1 = pltpu.CompilerParams(needs_layout_passes=False,
                                   use_tc_tiling_on_sc=False)
    phase1 = pl.kernel(
        _sc_phase1_body,
        out_type=[jax.ShapeDtypeStruct((E,), jnp.float32),
                  jax.ShapeDtypeStruct((NC, NPAD), jnp.float32)],
        mesh=mesh,
        compiler_params=params1,
        scratch_types=[
            pltpu.VMEM((N,), jnp.float32),
            pltpu.VMEM((N,), jnp.float32),
            pltpu.VMEM((EPT,), jnp.int32),
            pltpu.VMEM((EPT,), jnp.int32),
            pltpu.VMEM((EPT,), jnp.float32),
            pltpu.VMEM((EPT,), jnp.float32),
            pltpu.VMEM((NPAD,), jnp.float32),
            pltpu.VMEM((NS, CH), jnp.float32),
            pltpu.VMEM((CH,), jnp.float32),
            pltpu.VMEM_SHARED((NS, NPAD), jnp.float32),
        ],
    )
    phase1b = pl.kernel(
        _sc_phase1b_body,
        out_type=[jax.ShapeDtypeStruct((E,), jnp.float32)],
        mesh=mesh,
        compiler_params=params1,
        scratch_types=[
            pltpu.VMEM((EPT,), jnp.int32),
            pltpu.VMEM((EPT,), jnp.float32),
            pltpu.VMEM((EPT,), jnp.float32),
            pltpu.VMEM((NPAD,), jnp.float32),
            pltpu.VMEM((NPAD,), jnp.float32),
        ],
    )
    phase2 = pl.kernel(
        _sc_phase2_body,
        out_type=[jax.ShapeDtypeStruct((C * N,), jnp.float32)],
        mesh=mesh,
        compiler_params=params1,
        scratch_types=[
            pltpu.VMEM((TW * N,), jnp.float32),
            pltpu.VMEM((TW * N,), jnp.float32),
            pltpu.VMEM((CH2,), jnp.int32),
            pltpu.VMEM((CH2,), jnp.int32),
            pltpu.VMEM((CH2,), jnp.float32),
        ],
    )
    return phase1, phase1b, phase2


def kernel(x_src, x_dst, edge_index, edge_attr, W_src, W_edge, att_src,
           att_dst, att_edge, bias):
    src = edge_index[0]
    dst = edge_index[1]

    hst, hd, als, ald = pl.pallas_call(
        _tc_nodes_body,
        out_shape=[
            jax.ShapeDtypeStruct((C, N), jnp.float32),
            jax.ShapeDtypeStruct((N, C), jnp.float32),
            jax.ShapeDtypeStruct((1, N), jnp.float32),
            jax.ShapeDtypeStruct((N, 1), jnp.float32),
        ],
    )(x_src, x_dst, W_src, att_src, att_dst)

    EB = 20000
    ae = pl.pallas_call(
        _tc_edge_body,
        grid=(E // EB,),
        in_specs=[
            pl.BlockSpec((EB, C), lambda i: (i, 0)),
            pl.BlockSpec((C, C), lambda i: (0, 0)),
            pl.BlockSpec((1, C), lambda i: (0, 0)),
        ],
        out_specs=pl.BlockSpec((EB, 1), lambda i: (i, 0)),
        out_shape=jax.ShapeDtypeStruct((E, 1), jnp.float32),
    )(edge_attr, W_edge, att_edge)

    phase1, phase1b, phase2 = _build_sc_kernels()
    ex, den2 = phase1(als.reshape(N), ald.reshape(N), ae.reshape(E), src, dst)
    exd = jnp.concatenate([den2.reshape(NC * NPAD), ex])
    coef, = phase1b(dst, exd)
    partf, = phase2(hst.reshape(C * N), src, dst, coef)

    out = pl.pallas_call(
        _tc_comb_body,
        out_shape=jax.ShapeDtypeStruct((N, C), jnp.float32),
    )(partf.reshape(C, N), hd, bias.reshape(1, C))
    return out
